# baseline (speedup 1.0000x reference)
import functools

import jax
import jax.numpy as jnp
from jax import lax
from jax.experimental import pallas as pl
from jax.experimental.pallas import tpu as pltpu
from jax.experimental.pallas import tpu_sc as plsc

N = 10000
NPAD = 10112
E_RAW = 320000
E_TOT = E_RAW + N
NC, NS, NW = 2, 16, 32
CHUNK = 128
NCHUNK = 84
EPAD = NW * NCHUNK * CHUNK
ROWS_PER_TILE = NPAD // NS
Z1, Z2 = 312, 320

_MESH = dict(core_axis_name="c", subcore_axis_name="s", num_cores=NC,
             num_subcores=NS)


def _tca_body(x_ref, w1_ref, asm_ref, adm_ref,
              hg0_ref, hg1_ref, hg2_ref, hg3_ref, as1t_ref, ad1t_ref,
              m1_ref):
    h = jnp.dot(x_ref[...], w1_ref[...], preferred_element_type=jnp.float32)
    hg0_ref[...] = h[:, 0:64]
    hg1_ref[...] = h[:, 64:128]
    hg2_ref[...] = h[:, 128:192]
    hg3_ref[...] = h[:, 192:256]
    as1 = jnp.dot(h, asm_ref[...], preferred_element_type=jnp.float32)
    ad1 = jnp.dot(h, adm_ref[...], preferred_element_type=jnp.float32)
    as1t_ref[...] = jnp.concatenate([as1, as1], axis=1)
    ad1t_ref[...] = jnp.concatenate([ad1, ad1], axis=1)
    m = (jnp.max(as1, axis=0, keepdims=True)
         + jnp.max(ad1, axis=0, keepdims=True))
    m = jnp.where(m >= 0, m, 0.2 * m)
    m1_ref[...] = jnp.concatenate([m, m], axis=1)


TCB_BLOCKS = 8
TCB_B = NPAD // TCB_BLOCKS


def _tcb_body(u0_ref, u1_ref, u2_ref, u3_ref, d1_ref, b1_ref, w2_ref,
              r8_ref, as2m_ref, ad2m_ref, h2_ref, as2t_ref, ad2t_ref,
              m2_ref, mscr):
    i = pl.program_id(0)
    d8 = (d1_ref[0] + d1_ref[1])[:, :8]
    rec = 1.0 / (d8 + 1e-16)
    recx = jnp.dot(rec, r8_ref[...], preferred_element_type=jnp.float32)
    u = jnp.concatenate([u0_ref[0] + u0_ref[1], u1_ref[0] + u1_ref[1],
                         u2_ref[0] + u2_ref[1], u3_ref[0] + u3_ref[1]],
                        axis=1)
    h1 = u * recx + b1_ref[...]
    h1 = jnp.where(h1 > 0, h1, jnp.exp(jnp.minimum(h1, 0.0)) - 1.0)
    h2 = jnp.dot(h1, w2_ref[...], preferred_element_type=jnp.float32)
    h2_ref[...] = h2
    as2t = jnp.dot(h2, as2m_ref[...], preferred_element_type=jnp.float32)
    ad2t = jnp.dot(h2, ad2m_ref[...], preferred_element_type=jnp.float32)
    as2t_ref[...] = as2t
    ad2t_ref[...] = ad2t

    @pl.when(i == 0)
    def _():
        mscr[...] = jnp.full((2, 16), -jnp.inf, jnp.float32)

    bs = jnp.maximum(mscr[0:1, :], jnp.max(as2t, axis=0, keepdims=True))
    bd = jnp.maximum(mscr[1:2, :], jnp.max(ad2t, axis=0, keepdims=True))
    mscr[0:1, :] = bs
    mscr[1:2, :] = bd

    @pl.when(i == TCB_BLOCKS - 1)
    def _():
        m = mscr[0:1, :] + mscr[1:2, :]
        m2_ref[...] = jnp.where(m >= 0, m, 0.2 * m)


def _tcc_body(u2_ref, d2_ref, b2_ref, out_ref):
    out_ref[...] = ((u2_ref[0] + u2_ref[1])
                    / (d2_ref[0] + d2_ref[1] + 1e-16) + b2_ref[...])


def _sc1_body(srcr, dstr, as1t, ad1t, hg0, hg1, hg2, hg3, m1,
              u0o, u1o, u2o, u3o, d1o,
              sidx, didx, sA, sB, dA, dB, pbuf, hA, hB, m2v, zu, zd,
              accu, accd, asem, dsem, hsem):
    cid = lax.axis_index("c")
    sid = lax.axis_index("s")
    w = cid * NS + sid
    pltpu.sync_copy(srcr.at[w], sidx)
    pltpu.sync_copy(dstr.at[w], didx)
    pltpu.sync_copy(m1.at[0], m2v)
    mvec = m2v[...]
    zeros16 = jnp.zeros((16,), jnp.float32)

    def _zu(r, _):
        for k in range(4):
            zu[r, pl.ds(k * 16, 16)] = zeros16
        return 0
    lax.fori_loop(0, Z2, _zu, 0)

    def _zd(r, _):
        zd[r, :] = zeros16
        return 0
    lax.fori_loop(0, ROWS_PER_TILE, _zd, 0)

    r0 = sid * ROWS_PER_TILE
    pltpu.sync_copy(zu.at[pl.ds(0, Z1)], accu.at[pl.ds(r0, Z1)])
    pltpu.sync_copy(zu, accu.at[pl.ds(r0 + Z1, Z2)])
    pltpu.sync_copy(zd, accd.at[pl.ds(r0, ROWS_PER_TILE)])
    plsc.subcore_barrier()

    for g in range(4):
        hg = (hg0, hg1, hg2, hg3)[g]

        def _chunk(j, _):
            pltpu.sync_copy(as1t.at[sidx.at[j]], sA)
            pltpu.sync_copy(ad1t.at[didx.at[j]], dA)
            pltpu.sync_copy(hg.at[sidx.at[j]], hA)

            def _body(c, _c):
                e = sA[c, :] + dA[c, :]
                e = jnp.where(e >= 0, e, 0.2 * e)
                prow = jnp.exp(e - mvec)
                if g == 0:
                    pbuf[c, :] = prow
                for h in range(2):
                    vec = jnp.full((16,), prow[g * 2 + h], jnp.float32)
                    for half in range(2):
                        sl = pl.ds(h * 32 + half * 16, 16)
                        hA[c, sl] = hA[c, sl] * vec
                return 0
            lax.fori_loop(0, CHUNK, _body, 0)
            if g == 0:
                pltpu.sync_copy(pbuf, accd.at[didx.at[j]], add=True)
            pltpu.sync_copy(hA, accu.at[didx.at[j]], add=True)
            return 0
        lax.fori_loop(0, NCHUNK, _chunk, 0)
        plsc.subcore_barrier()

        uo = (u0o, u1o, u2o, u3o)[g]
        pltpu.sync_copy(accu.at[pl.ds(r0, Z1)],
                        uo.at[cid, pl.ds(r0, Z1)])
        pltpu.sync_copy(accu.at[pl.ds(r0 + Z1, Z2)],
                        uo.at[cid, pl.ds(r0 + Z1, Z2)])
        if g == 0:
            pltpu.sync_copy(accd.at[pl.ds(r0, ROWS_PER_TILE)],
                            d1o.at[cid, pl.ds(r0, ROWS_PER_TILE)])
        if g < 3:
            pltpu.sync_copy(zu.at[pl.ds(0, Z1)], accu.at[pl.ds(r0, Z1)])
            pltpu.sync_copy(zu, accu.at[pl.ds(r0 + Z1, Z2)])
            plsc.subcore_barrier()


def _sc2_body(srcr, dstr, h2t, as2t, ad2t, m2,
              u2o, d2o,
              sidx, didx, sA, sB, dA, dB, pbuf, hA, hB, m2v, zd,
              accu, accd, asem, dsem, hsem):
    cid = lax.axis_index("c")
    sid = lax.axis_index("s")
    w = cid * NS + sid
    pltpu.sync_copy(srcr.at[w], sidx)
    pltpu.sync_copy(dstr.at[w], didx)
    pltpu.sync_copy(m2.at[0], m2v)
    mvec = m2v[...]
    zeros16 = jnp.zeros((16,), jnp.float32)

    def _zd(r, _):
        zd[r, :] = zeros16
        return 0
    lax.fori_loop(0, ROWS_PER_TILE, _zd, 0)

    r0 = sid * ROWS_PER_TILE
    pltpu.sync_copy(zd, accu.at[pl.ds(r0, ROWS_PER_TILE)])
    pltpu.sync_copy(zd, accd.at[pl.ds(r0, ROWS_PER_TILE)])
    plsc.subcore_barrier()

    def _chunk(j, _):
        pltpu.sync_copy(as2t.at[sidx.at[j]], sA)
        pltpu.sync_copy(ad2t.at[didx.at[j]], dA)
        pltpu.sync_copy(h2t.at[sidx.at[j]], hA)

        def _body(c, _c):
            e = sA[c, :] + dA[c, :]
            e = jnp.where(e >= 0, e, 0.2 * e)
            prow = jnp.exp(e - mvec)
            pbuf[c, :] = prow
            hA[c, :] = hA[c, :] * prow
            return 0
        lax.fori_loop(0, CHUNK, _body, 0)
        pltpu.sync_copy(pbuf, accd.at[didx.at[j]], add=True)
        pltpu.sync_copy(hA, accu.at[didx.at[j]], add=True)
        return 0
    lax.fori_loop(0, NCHUNK, _chunk, 0)
    plsc.subcore_barrier()

    pltpu.sync_copy(accu.at[pl.ds(r0, ROWS_PER_TILE)],
                    u2o.at[cid, pl.ds(r0, ROWS_PER_TILE)])
    pltpu.sync_copy(accd.at[pl.ds(r0, ROWS_PER_TILE)],
                    d2o.at[cid, pl.ds(r0, ROWS_PER_TILE)])


def kernel(x, edge_index, W1, a_src1, a_dst1, b1, W2, a_src2, a_dst2, b2):
    f32 = jnp.float32
    loop = jnp.arange(N, dtype=jnp.int32)
    padi = jnp.full((EPAD - E_TOT,), N, jnp.int32)
    src = jnp.concatenate([edge_index[0].astype(jnp.int32), loop, padi])
    dst = jnp.concatenate([edge_index[1].astype(jnp.int32), loop, padi])
    srcr = src.reshape(NW, NCHUNK, CHUNK)
    dstr = dst.reshape(NW, NCHUNK, CHUNK)

    x_pad = jnp.zeros((NPAD, 128), f32).at[:N].set(x)
    eye8 = jnp.eye(8, dtype=f32)
    asm = (a_src1[:, :, None] * eye8[:, None, :]).reshape(256, 8)
    adm = (a_dst1[:, :, None] * eye8[:, None, :]).reshape(256, 8)
    r8 = jnp.kron(eye8, jnp.ones((1, 32), f32))
    as2m = jnp.broadcast_to(a_src2.reshape(16, 1), (16, 16))
    ad2m = jnp.broadcast_to(a_dst2.reshape(16, 1), (16, 16))

    hg0, hg1, hg2, hg3, as1t, ad1t, m1 = pl.pallas_call(
        _tca_body,
        out_shape=[
            jax.ShapeDtypeStruct((NPAD, 64), f32),
            jax.ShapeDtypeStruct((NPAD, 64), f32),
            jax.ShapeDtypeStruct((NPAD, 64), f32),
            jax.ShapeDtypeStruct((NPAD, 64), f32),
            jax.ShapeDtypeStruct((NPAD, 16), f32),
            jax.ShapeDtypeStruct((NPAD, 16), f32),
            jax.ShapeDtypeStruct((1, 16), f32),
        ],
    )(x_pad, W1, asm, adm)

    sc1 = functools.partial(
        pl.kernel,
        out_type=[
            jax.ShapeDtypeStruct((NC, NPAD, 64), f32),
            jax.ShapeDtypeStruct((NC, NPAD, 64), f32),
            jax.ShapeDtypeStruct((NC, NPAD, 64), f32),
            jax.ShapeDtypeStruct((NC, NPAD, 64), f32),
            jax.ShapeDtypeStruct((NC, NPAD, 16), f32),
        ],
        mesh=plsc.VectorSubcoreMesh(**_MESH),
        compiler_params=pltpu.CompilerParams(use_tc_tiling_on_sc=False),
        scratch_types=[
            pltpu.VMEM((NCHUNK, CHUNK), jnp.int32),
            pltpu.VMEM((NCHUNK, CHUNK), jnp.int32),
            pltpu.VMEM((CHUNK, 16), f32),
            pltpu.VMEM((CHUNK, 16), f32),
            pltpu.VMEM((CHUNK, 16), f32),
            pltpu.VMEM((CHUNK, 16), f32),
            pltpu.VMEM((CHUNK, 16), f32),
            pltpu.VMEM((CHUNK, 64), f32),
            pltpu.VMEM((CHUNK, 64), f32),
            pltpu.VMEM((16,), f32),
            pltpu.VMEM((Z2, 64), f32),
            pltpu.VMEM((ROWS_PER_TILE, 16), f32),
            pltpu.VMEM_SHARED((NPAD, 64), f32),
            pltpu.VMEM_SHARED((NPAD, 16), f32),
            pltpu.SemaphoreType.DMA,
            pltpu.SemaphoreType.DMA,
            pltpu.SemaphoreType.DMA,
        ],
    )(_sc1_body)
    u0, u1, u2, u3, d1 = sc1(srcr, dstr, as1t, ad1t, hg0, hg1, hg2, hg3,
                             m1)

    ublock = pl.BlockSpec((NC, TCB_B, 64), lambda i: (0, i, 0))
    h2t, as2t, ad2t, m2 = pl.pallas_call(
        _tcb_body,
        grid=(TCB_BLOCKS,),
        in_specs=[
            ublock, ublock, ublock, ublock,
            pl.BlockSpec((NC, TCB_B, 16), lambda i: (0, i, 0)),
            pl.BlockSpec((256,), lambda i: (0,)),
            pl.BlockSpec((256, 16), lambda i: (0, 0)),
            pl.BlockSpec((8, 256), lambda i: (0, 0)),
            pl.BlockSpec((16, 16), lambda i: (0, 0)),
            pl.BlockSpec((16, 16), lambda i: (0, 0)),
        ],
        out_specs=[
            pl.BlockSpec((TCB_B, 16), lambda i: (i, 0)),
            pl.BlockSpec((TCB_B, 16), lambda i: (i, 0)),
            pl.BlockSpec((TCB_B, 16), lambda i: (i, 0)),
            pl.BlockSpec((1, 16), lambda i: (0, 0)),
        ],
        out_shape=[
            jax.ShapeDtypeStruct((NPAD, 16), f32),
            jax.ShapeDtypeStruct((NPAD, 16), f32),
            jax.ShapeDtypeStruct((NPAD, 16), f32),
            jax.ShapeDtypeStruct((1, 16), f32),
        ],
        scratch_shapes=[pltpu.VMEM((2, 16), f32)],
    )(u0, u1, u2, u3, d1, b1, W2, r8, as2m, ad2m)

    sc2 = functools.partial(
        pl.kernel,
        out_type=[
            jax.ShapeDtypeStruct((NC, NPAD, 16), f32),
            jax.ShapeDtypeStruct((NC, NPAD, 16), f32),
        ],
        mesh=plsc.VectorSubcoreMesh(**_MESH),
        compiler_params=pltpu.CompilerParams(use_tc_tiling_on_sc=False),
        scratch_types=[
            pltpu.VMEM((NCHUNK, CHUNK), jnp.int32),
            pltpu.VMEM((NCHUNK, CHUNK), jnp.int32),
            pltpu.VMEM((CHUNK, 16), f32),
            pltpu.VMEM((CHUNK, 16), f32),
            pltpu.VMEM((CHUNK, 16), f32),
            pltpu.VMEM((CHUNK, 16), f32),
            pltpu.VMEM((CHUNK, 16), f32),
            pltpu.VMEM((CHUNK, 16), f32),
            pltpu.VMEM((CHUNK, 16), f32),
            pltpu.VMEM((16,), f32),
            pltpu.VMEM((ROWS_PER_TILE, 16), f32),
            pltpu.VMEM_SHARED((NPAD, 16), f32),
            pltpu.VMEM_SHARED((NPAD, 16), f32),
            pltpu.SemaphoreType.DMA,
            pltpu.SemaphoreType.DMA,
            pltpu.SemaphoreType.DMA,
        ],
    )(_sc2_body)
    u2, d2 = sc2(srcr, dstr, h2t, as2t, ad2t, m2)

    out = pl.pallas_call(
        _tcc_body,
        out_shape=jax.ShapeDtypeStruct((NPAD, 16), f32),
    )(u2, d2, b2)
    return out[:N]


# speedup vs baseline: 1.5851x; 1.5851x over previous
import functools

import jax
import jax.numpy as jnp
from jax import lax
from jax.experimental import pallas as pl
from jax.experimental.pallas import tpu as pltpu
from jax.experimental.pallas import tpu_sc as plsc

N = 10000
NPAD = 10112
E_RAW = 320000
E_TOT = E_RAW + N
NC, NS, NW = 2, 16, 32
CHUNK = 128
NCHUNK = 84
EPAD = NW * NCHUNK * CHUNK
ROWS_PER_TILE = NPAD // NS
Z1, Z2 = 312, 320

_MESH = dict(core_axis_name="c", subcore_axis_name="s", num_cores=NC,
             num_subcores=NS)


def _tca_body(x_ref, w1_ref, asm_ref, adm_ref,
              hg0_ref, hg1_ref, hg2_ref, hg3_ref, as1t_ref, ad1t_ref,
              m1_ref):
    h = jnp.dot(x_ref[...], w1_ref[...], preferred_element_type=jnp.float32)
    hg0_ref[...] = h[:, 0:64]
    hg1_ref[...] = h[:, 64:128]
    hg2_ref[...] = h[:, 128:192]
    hg3_ref[...] = h[:, 192:256]
    as1 = jnp.dot(h, asm_ref[...], preferred_element_type=jnp.float32)
    ad1 = jnp.dot(h, adm_ref[...], preferred_element_type=jnp.float32)
    as1t_ref[...] = jnp.concatenate([as1, as1], axis=1)
    ad1t_ref[...] = jnp.concatenate([ad1, ad1], axis=1)
    m = (jnp.max(as1, axis=0, keepdims=True)
         + jnp.max(ad1, axis=0, keepdims=True))
    m = jnp.where(m >= 0, m, 0.2 * m)
    m1_ref[...] = jnp.concatenate([m, m], axis=1)


TCB_BLOCKS = 8
TCB_B = NPAD // TCB_BLOCKS


def _tcb_body(u0_ref, u1_ref, u2_ref, u3_ref, d1_ref, b1_ref, w2_ref,
              r8_ref, as2m_ref, ad2m_ref, h2_ref, as2t_ref, ad2t_ref,
              m2_ref, mscr):
    i = pl.program_id(0)
    d8 = (d1_ref[0] + d1_ref[1])[:, :8]
    rec = 1.0 / (d8 + 1e-16)
    recx = jnp.dot(rec, r8_ref[...], preferred_element_type=jnp.float32)
    u = jnp.concatenate([u0_ref[0] + u0_ref[1], u1_ref[0] + u1_ref[1],
                         u2_ref[0] + u2_ref[1], u3_ref[0] + u3_ref[1]],
                        axis=1)
    h1 = u * recx + b1_ref[...]
    h1 = jnp.where(h1 > 0, h1, jnp.exp(jnp.minimum(h1, 0.0)) - 1.0)
    h2 = jnp.dot(h1, w2_ref[...], preferred_element_type=jnp.float32)
    h2_ref[...] = h2
    as2t = jnp.dot(h2, as2m_ref[...], preferred_element_type=jnp.float32)
    ad2t = jnp.dot(h2, ad2m_ref[...], preferred_element_type=jnp.float32)
    as2t_ref[...] = as2t
    ad2t_ref[...] = ad2t

    @pl.when(i == 0)
    def _():
        mscr[...] = jnp.full((2, 16), -jnp.inf, jnp.float32)

    bs = jnp.maximum(mscr[0:1, :], jnp.max(as2t, axis=0, keepdims=True))
    bd = jnp.maximum(mscr[1:2, :], jnp.max(ad2t, axis=0, keepdims=True))
    mscr[0:1, :] = bs
    mscr[1:2, :] = bd

    @pl.when(i == TCB_BLOCKS - 1)
    def _():
        m = mscr[0:1, :] + mscr[1:2, :]
        m2_ref[...] = jnp.where(m >= 0, m, 0.2 * m)


def _tcc_body(u2_ref, d2_ref, b2_ref, out_ref):
    out_ref[...] = ((u2_ref[0] + u2_ref[1])
                    / (d2_ref[0] + d2_ref[1] + 1e-16) + b2_ref[...])


def _sc1_body(srcr, dstr, as1t, ad1t, hg0, hg1, hg2, hg3, m1,
              u0o, u1o, u2o, u3o, d1o,
              sidx, didx, sA, sB, dA, dB, pbuf, hA, hB, m2v, zu, zd,
              accu, accd, asem, dsem, hsem):
    cid = lax.axis_index("c")
    sid = lax.axis_index("s")
    w = cid * NS + sid
    pltpu.sync_copy(srcr.at[w], sidx)
    pltpu.sync_copy(dstr.at[w], didx)
    pltpu.sync_copy(m1.at[0], m2v)
    mvec = m2v[...]
    zeros16 = jnp.zeros((16,), jnp.float32)

    def _zu(r, _):
        for k in range(4):
            zu[r, pl.ds(k * 16, 16)] = zeros16
        return 0
    lax.fori_loop(0, Z2, _zu, 0)

    def _zd(r, _):
        zd[r, :] = zeros16
        return 0
    lax.fori_loop(0, ROWS_PER_TILE, _zd, 0)

    r0 = sid * ROWS_PER_TILE
    pltpu.sync_copy(zu.at[pl.ds(0, Z1)], accu.at[pl.ds(r0, Z1)])
    pltpu.sync_copy(zu, accu.at[pl.ds(r0 + Z1, Z2)])
    pltpu.sync_copy(zd, accd.at[pl.ds(r0, ROWS_PER_TILE)])
    plsc.subcore_barrier()

    for g in range(4):
        hg = (hg0, hg1, hg2, hg3)[g]

        def _start(j, sb, db, hb):
            pltpu.async_copy(as1t.at[sidx.at[j]], sb, asem)
            pltpu.async_copy(ad1t.at[didx.at[j]], db, dsem)
            pltpu.async_copy(hg.at[sidx.at[j]], hb, hsem)

        def _wait(j, sb, db, hb):
            pltpu.make_async_copy(as1t.at[sidx.at[j]], sb, asem).wait()
            pltpu.make_async_copy(ad1t.at[didx.at[j]], db, dsem).wait()
            pltpu.make_async_copy(hg.at[sidx.at[j]], hb, hsem).wait()

        def _edges(j, sb, db, hb):
            def _pcalc(c, _c):
                e = sb[c, :] + db[c, :]
                e = jnp.where(e >= 0, e, 0.2 * e)
                pbuf[c, :] = jnp.exp(e - mvec)
                return 0
            lax.fori_loop(0, CHUNK, _pcalc, 0)
            if g == 0:
                pltpu.sync_copy(pbuf, accd.at[didx.at[j]], add=True)

            def _mul(c, _c):
                prow = pbuf[c, :]
                for h in range(2):
                    vec = jnp.full((16,), prow[g * 2 + h], jnp.float32)
                    for half in range(2):
                        sl = pl.ds(h * 32 + half * 16, 16)
                        hb[c, sl] = hb[c, sl] * vec
                return 0
            lax.fori_loop(0, CHUNK, _mul, 0)
            pltpu.sync_copy(hb, accu.at[didx.at[j]], add=True)

        _start(0, sA, dA, hA)

        def _pair(jj, _):
            j0 = 2 * jj
            _wait(j0, sA, dA, hA)
            _start(j0 + 1, sB, dB, hB)
            _edges(j0, sA, dA, hA)
            _wait(j0 + 1, sB, dB, hB)

            @pl.when(jj < NCHUNK // 2 - 1)
            def _():
                _start(j0 + 2, sA, dA, hA)
            _edges(j0 + 1, sB, dB, hB)
            return 0
        lax.fori_loop(0, NCHUNK // 2, _pair, 0)
        plsc.subcore_barrier()

        uo = (u0o, u1o, u2o, u3o)[g]
        pltpu.sync_copy(accu.at[pl.ds(r0, Z1)],
                        uo.at[cid, pl.ds(r0, Z1)])
        pltpu.sync_copy(accu.at[pl.ds(r0 + Z1, Z2)],
                        uo.at[cid, pl.ds(r0 + Z1, Z2)])
        if g == 0:
            pltpu.sync_copy(accd.at[pl.ds(r0, ROWS_PER_TILE)],
                            d1o.at[cid, pl.ds(r0, ROWS_PER_TILE)])
        if g < 3:
            pltpu.sync_copy(zu.at[pl.ds(0, Z1)], accu.at[pl.ds(r0, Z1)])
            pltpu.sync_copy(zu, accu.at[pl.ds(r0 + Z1, Z2)])
            plsc.subcore_barrier()


def _sc2_body(srcr, dstr, h2t, as2t, ad2t, m2,
              u2o, d2o,
              sidx, didx, sA, sB, dA, dB, pbuf, hA, hB, m2v, zd,
              accu, accd, asem, dsem, hsem):
    cid = lax.axis_index("c")
    sid = lax.axis_index("s")
    w = cid * NS + sid
    pltpu.sync_copy(srcr.at[w], sidx)
    pltpu.sync_copy(dstr.at[w], didx)
    pltpu.sync_copy(m2.at[0], m2v)
    mvec = m2v[...]
    zeros16 = jnp.zeros((16,), jnp.float32)

    def _zd(r, _):
        zd[r, :] = zeros16
        return 0
    lax.fori_loop(0, ROWS_PER_TILE, _zd, 0)

    r0 = sid * ROWS_PER_TILE
    pltpu.sync_copy(zd, accu.at[pl.ds(r0, ROWS_PER_TILE)])
    pltpu.sync_copy(zd, accd.at[pl.ds(r0, ROWS_PER_TILE)])
    plsc.subcore_barrier()

    def _start(j, sb, db, hb):
        pltpu.async_copy(as2t.at[sidx.at[j]], sb, asem)
        pltpu.async_copy(ad2t.at[didx.at[j]], db, dsem)
        pltpu.async_copy(h2t.at[sidx.at[j]], hb, hsem)

    def _wait(j, sb, db, hb):
        pltpu.make_async_copy(as2t.at[sidx.at[j]], sb, asem).wait()
        pltpu.make_async_copy(ad2t.at[didx.at[j]], db, dsem).wait()
        pltpu.make_async_copy(h2t.at[sidx.at[j]], hb, hsem).wait()

    def _edges(j, sb, db, hb):
        def _pcalc(c, _c):
            e = sb[c, :] + db[c, :]
            e = jnp.where(e >= 0, e, 0.2 * e)
            pbuf[c, :] = jnp.exp(e - mvec)
            return 0
        lax.fori_loop(0, CHUNK, _pcalc, 0)
        pltpu.sync_copy(pbuf, accd.at[didx.at[j]], add=True)

        def _mul(c, _c):
            hb[c, :] = hb[c, :] * pbuf[c, :]
            return 0
        lax.fori_loop(0, CHUNK, _mul, 0)
        pltpu.sync_copy(hb, accu.at[didx.at[j]], add=True)

    _start(0, sA, dA, hA)

    def _pair(jj, _):
        j0 = 2 * jj
        _wait(j0, sA, dA, hA)
        _start(j0 + 1, sB, dB, hB)
        _edges(j0, sA, dA, hA)
        _wait(j0 + 1, sB, dB, hB)

        @pl.when(jj < NCHUNK // 2 - 1)
        def _():
            _start(j0 + 2, sA, dA, hA)
        _edges(j0 + 1, sB, dB, hB)
        return 0
    lax.fori_loop(0, NCHUNK // 2, _pair, 0)
    plsc.subcore_barrier()

    pltpu.sync_copy(accu.at[pl.ds(r0, ROWS_PER_TILE)],
                    u2o.at[cid, pl.ds(r0, ROWS_PER_TILE)])
    pltpu.sync_copy(accd.at[pl.ds(r0, ROWS_PER_TILE)],
                    d2o.at[cid, pl.ds(r0, ROWS_PER_TILE)])


def kernel(x, edge_index, W1, a_src1, a_dst1, b1, W2, a_src2, a_dst2, b2):
    f32 = jnp.float32
    loop = jnp.arange(N, dtype=jnp.int32)
    padi = jnp.full((EPAD - E_TOT,), N, jnp.int32)
    src = jnp.concatenate([edge_index[0].astype(jnp.int32), loop, padi])
    dst = jnp.concatenate([edge_index[1].astype(jnp.int32), loop, padi])
    srcr = src.reshape(NW, NCHUNK, CHUNK)
    dstr = dst.reshape(NW, NCHUNK, CHUNK)

    x_pad = jnp.zeros((NPAD, 128), f32).at[:N].set(x)
    eye8 = jnp.eye(8, dtype=f32)
    asm = (a_src1[:, :, None] * eye8[:, None, :]).reshape(256, 8)
    adm = (a_dst1[:, :, None] * eye8[:, None, :]).reshape(256, 8)
    r8 = jnp.kron(eye8, jnp.ones((1, 32), f32))
    as2m = jnp.broadcast_to(a_src2.reshape(16, 1), (16, 16))
    ad2m = jnp.broadcast_to(a_dst2.reshape(16, 1), (16, 16))

    hg0, hg1, hg2, hg3, as1t, ad1t, m1 = pl.pallas_call(
        _tca_body,
        out_shape=[
            jax.ShapeDtypeStruct((NPAD, 64), f32),
            jax.ShapeDtypeStruct((NPAD, 64), f32),
            jax.ShapeDtypeStruct((NPAD, 64), f32),
            jax.ShapeDtypeStruct((NPAD, 64), f32),
            jax.ShapeDtypeStruct((NPAD, 16), f32),
            jax.ShapeDtypeStruct((NPAD, 16), f32),
            jax.ShapeDtypeStruct((1, 16), f32),
        ],
    )(x_pad, W1, asm, adm)

    sc1 = functools.partial(
        pl.kernel,
        out_type=[
            jax.ShapeDtypeStruct((NC, NPAD, 64), f32),
            jax.ShapeDtypeStruct((NC, NPAD, 64), f32),
            jax.ShapeDtypeStruct((NC, NPAD, 64), f32),
            jax.ShapeDtypeStruct((NC, NPAD, 64), f32),
            jax.ShapeDtypeStruct((NC, NPAD, 16), f32),
        ],
        mesh=plsc.VectorSubcoreMesh(**_MESH),
        compiler_params=pltpu.CompilerParams(use_tc_tiling_on_sc=False),
        scratch_types=[
            pltpu.VMEM((NCHUNK, CHUNK), jnp.int32),
            pltpu.VMEM((NCHUNK, CHUNK), jnp.int32),
            pltpu.VMEM((CHUNK, 16), f32),
            pltpu.VMEM((CHUNK, 16), f32),
            pltpu.VMEM((CHUNK, 16), f32),
            pltpu.VMEM((CHUNK, 16), f32),
            pltpu.VMEM((CHUNK, 16), f32),
            pltpu.VMEM((CHUNK, 64), f32),
            pltpu.VMEM((CHUNK, 64), f32),
            pltpu.VMEM((16,), f32),
            pltpu.VMEM((Z2, 64), f32),
            pltpu.VMEM((ROWS_PER_TILE, 16), f32),
            pltpu.VMEM_SHARED((NPAD, 64), f32),
            pltpu.VMEM_SHARED((NPAD, 16), f32),
            pltpu.SemaphoreType.DMA,
            pltpu.SemaphoreType.DMA,
            pltpu.SemaphoreType.DMA,
        ],
    )(_sc1_body)
    u0, u1, u2, u3, d1 = sc1(srcr, dstr, as1t, ad1t, hg0, hg1, hg2, hg3,
                             m1)

    ublock = pl.BlockSpec((NC, TCB_B, 64), lambda i: (0, i, 0))
    h2t, as2t, ad2t, m2 = pl.pallas_call(
        _tcb_body,
        grid=(TCB_BLOCKS,),
        in_specs=[
            ublock, ublock, ublock, ublock,
            pl.BlockSpec((NC, TCB_B, 16), lambda i: (0, i, 0)),
            pl.BlockSpec((256,), lambda i: (0,)),
            pl.BlockSpec((256, 16), lambda i: (0, 0)),
            pl.BlockSpec((8, 256), lambda i: (0, 0)),
            pl.BlockSpec((16, 16), lambda i: (0, 0)),
            pl.BlockSpec((16, 16), lambda i: (0, 0)),
        ],
        out_specs=[
            pl.BlockSpec((TCB_B, 16), lambda i: (i, 0)),
            pl.BlockSpec((TCB_B, 16), lambda i: (i, 0)),
            pl.BlockSpec((TCB_B, 16), lambda i: (i, 0)),
            pl.BlockSpec((1, 16), lambda i: (0, 0)),
        ],
        out_shape=[
            jax.ShapeDtypeStruct((NPAD, 16), f32),
            jax.ShapeDtypeStruct((NPAD, 16), f32),
            jax.ShapeDtypeStruct((NPAD, 16), f32),
            jax.ShapeDtypeStruct((1, 16), f32),
        ],
        scratch_shapes=[pltpu.VMEM((2, 16), f32)],
    )(u0, u1, u2, u3, d1, b1, W2, r8, as2m, ad2m)

    sc2 = functools.partial(
        pl.kernel,
        out_type=[
            jax.ShapeDtypeStruct((NC, NPAD, 16), f32),
            jax.ShapeDtypeStruct((NC, NPAD, 16), f32),
        ],
        mesh=plsc.VectorSubcoreMesh(**_MESH),
        compiler_params=pltpu.CompilerParams(use_tc_tiling_on_sc=False),
        scratch_types=[
            pltpu.VMEM((NCHUNK, CHUNK), jnp.int32),
            pltpu.VMEM((NCHUNK, CHUNK), jnp.int32),
            pltpu.VMEM((CHUNK, 16), f32),
            pltpu.VMEM((CHUNK, 16), f32),
            pltpu.VMEM((CHUNK, 16), f32),
            pltpu.VMEM((CHUNK, 16), f32),
            pltpu.VMEM((CHUNK, 16), f32),
            pltpu.VMEM((CHUNK, 16), f32),
            pltpu.VMEM((CHUNK, 16), f32),
            pltpu.VMEM((16,), f32),
            pltpu.VMEM((ROWS_PER_TILE, 16), f32),
            pltpu.VMEM_SHARED((NPAD, 16), f32),
            pltpu.VMEM_SHARED((NPAD, 16), f32),
            pltpu.SemaphoreType.DMA,
            pltpu.SemaphoreType.DMA,
            pltpu.SemaphoreType.DMA,
        ],
    )(_sc2_body)
    u2, d2 = sc2(srcr, dstr, h2t, as2t, ad2t, m2)

    out = pl.pallas_call(
        _tcc_body,
        out_shape=jax.ShapeDtypeStruct((NPAD, 16), f32),
    )(u2, d2, b2)
    return out[:N]


# speedup vs baseline: 2.2608x; 1.4263x over previous
import functools

import jax
import jax.numpy as jnp
from jax import lax
from jax.experimental import pallas as pl
from jax.experimental.pallas import tpu as pltpu
from jax.experimental.pallas import tpu_sc as plsc

N = 10000
NPAD = 10112
E_RAW = 320000
E_TOT = E_RAW + N
NC, NS, NW = 2, 16, 32
CHUNK = 128
NCHUNK = 81
EPAD = NW * NCHUNK * CHUNK
ROWS_PER_TILE = NPAD // NS
Z1, Z2 = 312, 320

_MESH = dict(core_axis_name="c", subcore_axis_name="s", num_cores=NC,
             num_subcores=NS)


def _tca_body(x_ref, w1_ref, asm_ref, adm_ref,
              hg0_ref, hg1_ref, hg2_ref, hg3_ref, as1t_ref, ad1t_ref,
              m1_ref):
    h = jnp.dot(x_ref[...], w1_ref[...], preferred_element_type=jnp.float32)
    hg0_ref[...] = h[:, 0:64]
    hg1_ref[...] = h[:, 64:128]
    hg2_ref[...] = h[:, 128:192]
    hg3_ref[...] = h[:, 192:256]
    as1 = jnp.dot(h, asm_ref[...], preferred_element_type=jnp.float32)
    ad1 = jnp.dot(h, adm_ref[...], preferred_element_type=jnp.float32)
    as1t_ref[...] = jnp.concatenate([as1, as1], axis=1)
    ad1t_ref[...] = jnp.concatenate([ad1, ad1], axis=1)
    m = (jnp.max(as1, axis=0, keepdims=True)
         + jnp.max(ad1, axis=0, keepdims=True))
    m = jnp.where(m >= 0, m, 0.2 * m)
    m1_ref[...] = jnp.concatenate([m, m], axis=1)


TCB_BLOCKS = 8
TCB_B = NPAD // TCB_BLOCKS


def _tcb_body(u0_ref, u1_ref, u2_ref, u3_ref, d1_ref, b1_ref, w2_ref,
              r8_ref, as2m_ref, ad2m_ref, h2_ref, as2t_ref, ad2t_ref,
              m2_ref, mscr):
    i = pl.program_id(0)
    d8 = (d1_ref[0] + d1_ref[1])[:, :8]
    rec = 1.0 / (d8 + 1e-16)
    recx = jnp.dot(rec, r8_ref[...], preferred_element_type=jnp.float32)
    u = jnp.concatenate([u0_ref[0] + u0_ref[1], u1_ref[0] + u1_ref[1],
                         u2_ref[0] + u2_ref[1], u3_ref[0] + u3_ref[1]],
                        axis=1)
    h1 = u * recx + b1_ref[...]
    h1 = jnp.where(h1 > 0, h1, jnp.exp(jnp.minimum(h1, 0.0)) - 1.0)
    h2 = jnp.dot(h1, w2_ref[...], preferred_element_type=jnp.float32)
    h2_ref[...] = h2
    as2t = jnp.dot(h2, as2m_ref[...], preferred_element_type=jnp.float32)
    ad2t = jnp.dot(h2, ad2m_ref[...], preferred_element_type=jnp.float32)
    as2t_ref[...] = as2t
    ad2t_ref[...] = ad2t

    @pl.when(i == 0)
    def _():
        mscr[...] = jnp.full((2, 16), -jnp.inf, jnp.float32)

    bs = jnp.maximum(mscr[0:1, :], jnp.max(as2t, axis=0, keepdims=True))
    bd = jnp.maximum(mscr[1:2, :], jnp.max(ad2t, axis=0, keepdims=True))
    mscr[0:1, :] = bs
    mscr[1:2, :] = bd

    @pl.when(i == TCB_BLOCKS - 1)
    def _():
        m = mscr[0:1, :] + mscr[1:2, :]
        m2_ref[...] = jnp.where(m >= 0, m, 0.2 * m)


def _tcc_body(u2_ref, d2_ref, b2_ref, out_ref):
    out_ref[...] = ((u2_ref[0] + u2_ref[1])
                    / (d2_ref[0] + d2_ref[1] + 1e-16) + b2_ref[...])


def _sc1_body(srcr, dstr, as1t, ad1t, hg0, hg1, hg2, hg3, m1,
              u0o, u1o, u2o, u3o, d1o,
              sidx, didx, sA, sB, dA, dB, pbuf, hA, hB, m2v, zu, zd,
              accu, accd, asem, dsem, hsem):
    cid = lax.axis_index("c")
    sid = lax.axis_index("s")
    w = cid * NS + sid
    pltpu.sync_copy(srcr.at[w], sidx)
    pltpu.sync_copy(dstr.at[w], didx)
    pltpu.sync_copy(m1.at[0], m2v)
    mvec = m2v[...]
    zeros16 = jnp.zeros((16,), jnp.float32)

    def _zu(r, _):
        for k in range(4):
            zu[r, pl.ds(k * 16, 16)] = zeros16
        return 0
    lax.fori_loop(0, Z2, _zu, 0)

    def _zd(r, _):
        zd[r, :] = zeros16
        return 0
    lax.fori_loop(0, ROWS_PER_TILE, _zd, 0)

    r0 = sid * ROWS_PER_TILE
    pltpu.sync_copy(zu.at[pl.ds(0, Z1)], accu.at[pl.ds(r0, Z1)])
    pltpu.sync_copy(zu, accu.at[pl.ds(r0 + Z1, Z2)])
    pltpu.sync_copy(zd, accd.at[pl.ds(r0, ROWS_PER_TILE)])
    plsc.subcore_barrier()

    for g in range(4):
        hg = (hg0, hg1, hg2, hg3)[g]

        def _chunk(j, _):
            hd = pltpu.async_copy(hg.at[sidx.at[j]], hA, hsem)
            pltpu.sync_copy(as1t.at[sidx.at[j]], sA)
            pltpu.sync_copy(ad1t.at[didx.at[j]], dA)

            def _pcalc(c, _c):
                e = sA[c, :] + dA[c, :]
                e = jnp.where(e >= 0, e, 0.2 * e)
                pbuf[c, :] = jnp.exp(e - mvec)
                return 0
            lax.fori_loop(0, CHUNK, _pcalc, 0)
            if g == 0:
                pltpu.sync_copy(pbuf, accd.at[didx.at[j]], add=True)
            hd.wait()

            def _mul(c, _c):
                prow = pbuf[c, :]
                for h in range(2):
                    vec = jnp.full((16,), prow[g * 2 + h], jnp.float32)
                    for half in range(2):
                        sl = pl.ds(h * 32 + half * 16, 16)
                        hA[c, sl] = hA[c, sl] * vec
                return 0
            lax.fori_loop(0, CHUNK, _mul, 0)
            pltpu.sync_copy(hA, accu.at[didx.at[j]], add=True)
            return 0
        lax.fori_loop(0, NCHUNK, _chunk, 0)
        plsc.subcore_barrier()

        uo = (u0o, u1o, u2o, u3o)[g]
        pltpu.sync_copy(accu.at[pl.ds(r0, Z1)],
                        uo.at[cid, pl.ds(r0, Z1)])
        pltpu.sync_copy(accu.at[pl.ds(r0 + Z1, Z2)],
                        uo.at[cid, pl.ds(r0 + Z1, Z2)])
        if g == 0:
            pltpu.sync_copy(accd.at[pl.ds(r0, ROWS_PER_TILE)],
                            d1o.at[cid, pl.ds(r0, ROWS_PER_TILE)])
        if g < 3:
            pltpu.sync_copy(zu.at[pl.ds(0, Z1)], accu.at[pl.ds(r0, Z1)])
            pltpu.sync_copy(zu, accu.at[pl.ds(r0 + Z1, Z2)])
            plsc.subcore_barrier()


def _sc2_body(srcr, dstr, h2t, as2t, ad2t, m2,
              u2o, d2o,
              sidx, didx, sA, sB, dA, dB, pbuf, hA, hB, m2v, zd,
              accu, accd, asem, dsem, hsem):
    cid = lax.axis_index("c")
    sid = lax.axis_index("s")
    w = cid * NS + sid
    pltpu.sync_copy(srcr.at[w], sidx)
    pltpu.sync_copy(dstr.at[w], didx)
    pltpu.sync_copy(m2.at[0], m2v)
    mvec = m2v[...]
    zeros16 = jnp.zeros((16,), jnp.float32)

    def _zd(r, _):
        zd[r, :] = zeros16
        return 0
    lax.fori_loop(0, ROWS_PER_TILE, _zd, 0)

    r0 = sid * ROWS_PER_TILE
    pltpu.sync_copy(zd, accu.at[pl.ds(r0, ROWS_PER_TILE)])
    pltpu.sync_copy(zd, accd.at[pl.ds(r0, ROWS_PER_TILE)])
    plsc.subcore_barrier()

    def _chunk(j, _):
        hd = pltpu.async_copy(h2t.at[sidx.at[j]], hA, hsem)
        pltpu.sync_copy(as2t.at[sidx.at[j]], sA)
        pltpu.sync_copy(ad2t.at[didx.at[j]], dA)

        def _pcalc(c, _c):
            e = sA[c, :] + dA[c, :]
            e = jnp.where(e >= 0, e, 0.2 * e)
            pbuf[c, :] = jnp.exp(e - mvec)
            return 0
        lax.fori_loop(0, CHUNK, _pcalc, 0)
        pltpu.sync_copy(pbuf, accd.at[didx.at[j]], add=True)
        hd.wait()

        def _mul(c, _c):
            hA[c, :] = hA[c, :] * pbuf[c, :]
            return 0
        lax.fori_loop(0, CHUNK, _mul, 0)
        pltpu.sync_copy(hA, accu.at[didx.at[j]], add=True)
        return 0
    lax.fori_loop(0, NCHUNK, _chunk, 0)
    plsc.subcore_barrier()

    pltpu.sync_copy(accu.at[pl.ds(r0, ROWS_PER_TILE)],
                    u2o.at[cid, pl.ds(r0, ROWS_PER_TILE)])
    pltpu.sync_copy(accd.at[pl.ds(r0, ROWS_PER_TILE)],
                    d2o.at[cid, pl.ds(r0, ROWS_PER_TILE)])


def kernel(x, edge_index, W1, a_src1, a_dst1, b1, W2, a_src2, a_dst2, b2):
    f32 = jnp.float32
    loop = jnp.arange(N, dtype=jnp.int32)
    padi = jnp.full((EPAD - E_TOT,), N, jnp.int32)
    src = jnp.concatenate([edge_index[0].astype(jnp.int32), loop, padi])
    dst = jnp.concatenate([edge_index[1].astype(jnp.int32), loop, padi])
    srcr = src.reshape(NW, NCHUNK, CHUNK)
    dstr = dst.reshape(NW, NCHUNK, CHUNK)

    x_pad = jnp.zeros((NPAD, 128), f32).at[:N].set(x)
    eye8 = jnp.eye(8, dtype=f32)
    asm = (a_src1[:, :, None] * eye8[:, None, :]).reshape(256, 8)
    adm = (a_dst1[:, :, None] * eye8[:, None, :]).reshape(256, 8)
    r8 = jnp.kron(eye8, jnp.ones((1, 32), f32))
    as2m = jnp.broadcast_to(a_src2.reshape(16, 1), (16, 16))
    ad2m = jnp.broadcast_to(a_dst2.reshape(16, 1), (16, 16))

    hg0, hg1, hg2, hg3, as1t, ad1t, m1 = pl.pallas_call(
        _tca_body,
        out_shape=[
            jax.ShapeDtypeStruct((NPAD, 64), f32),
            jax.ShapeDtypeStruct((NPAD, 64), f32),
            jax.ShapeDtypeStruct((NPAD, 64), f32),
            jax.ShapeDtypeStruct((NPAD, 64), f32),
            jax.ShapeDtypeStruct((NPAD, 16), f32),
            jax.ShapeDtypeStruct((NPAD, 16), f32),
            jax.ShapeDtypeStruct((1, 16), f32),
        ],
    )(x_pad, W1, asm, adm)

    sc1 = functools.partial(
        pl.kernel,
        out_type=[
            jax.ShapeDtypeStruct((NC, NPAD, 64), f32),
            jax.ShapeDtypeStruct((NC, NPAD, 64), f32),
            jax.ShapeDtypeStruct((NC, NPAD, 64), f32),
            jax.ShapeDtypeStruct((NC, NPAD, 64), f32),
            jax.ShapeDtypeStruct((NC, NPAD, 16), f32),
        ],
        mesh=plsc.VectorSubcoreMesh(**_MESH),
        compiler_params=pltpu.CompilerParams(use_tc_tiling_on_sc=False),
        scratch_types=[
            pltpu.VMEM((NCHUNK, CHUNK), jnp.int32),
            pltpu.VMEM((NCHUNK, CHUNK), jnp.int32),
            pltpu.VMEM((CHUNK, 16), f32),
            pltpu.VMEM((CHUNK, 16), f32),
            pltpu.VMEM((CHUNK, 16), f32),
            pltpu.VMEM((CHUNK, 16), f32),
            pltpu.VMEM((CHUNK, 16), f32),
            pltpu.VMEM((CHUNK, 64), f32),
            pltpu.VMEM((CHUNK, 64), f32),
            pltpu.VMEM((16,), f32),
            pltpu.VMEM((Z2, 64), f32),
            pltpu.VMEM((ROWS_PER_TILE, 16), f32),
            pltpu.VMEM_SHARED((NPAD, 64), f32),
            pltpu.VMEM_SHARED((NPAD, 16), f32),
            pltpu.SemaphoreType.DMA,
            pltpu.SemaphoreType.DMA,
            pltpu.SemaphoreType.DMA,
        ],
    )(_sc1_body)
    u0, u1, u2, u3, d1 = sc1(srcr, dstr, as1t, ad1t, hg0, hg1, hg2, hg3,
                             m1)

    ublock = pl.BlockSpec((NC, TCB_B, 64), lambda i: (0, i, 0))
    h2t, as2t, ad2t, m2 = pl.pallas_call(
        _tcb_body,
        grid=(TCB_BLOCKS,),
        in_specs=[
            ublock, ublock, ublock, ublock,
            pl.BlockSpec((NC, TCB_B, 16), lambda i: (0, i, 0)),
            pl.BlockSpec((256,), lambda i: (0,)),
            pl.BlockSpec((256, 16), lambda i: (0, 0)),
            pl.BlockSpec((8, 256), lambda i: (0, 0)),
            pl.BlockSpec((16, 16), lambda i: (0, 0)),
            pl.BlockSpec((16, 16), lambda i: (0, 0)),
        ],
        out_specs=[
            pl.BlockSpec((TCB_B, 16), lambda i: (i, 0)),
            pl.BlockSpec((TCB_B, 16), lambda i: (i, 0)),
            pl.BlockSpec((TCB_B, 16), lambda i: (i, 0)),
            pl.BlockSpec((1, 16), lambda i: (0, 0)),
        ],
        out_shape=[
            jax.ShapeDtypeStruct((NPAD, 16), f32),
            jax.ShapeDtypeStruct((NPAD, 16), f32),
            jax.ShapeDtypeStruct((NPAD, 16), f32),
            jax.ShapeDtypeStruct((1, 16), f32),
        ],
        scratch_shapes=[pltpu.VMEM((2, 16), f32)],
    )(u0, u1, u2, u3, d1, b1, W2, r8, as2m, ad2m)

    sc2 = functools.partial(
        pl.kernel,
        out_type=[
            jax.ShapeDtypeStruct((NC, NPAD, 16), f32),
            jax.ShapeDtypeStruct((NC, NPAD, 16), f32),
        ],
        mesh=plsc.VectorSubcoreMesh(**_MESH),
        compiler_params=pltpu.CompilerParams(use_tc_tiling_on_sc=False),
        scratch_types=[
            pltpu.VMEM((NCHUNK, CHUNK), jnp.int32),
            pltpu.VMEM((NCHUNK, CHUNK), jnp.int32),
            pltpu.VMEM((CHUNK, 16), f32),
            pltpu.VMEM((CHUNK, 16), f32),
            pltpu.VMEM((CHUNK, 16), f32),
            pltpu.VMEM((CHUNK, 16), f32),
            pltpu.VMEM((CHUNK, 16), f32),
            pltpu.VMEM((CHUNK, 16), f32),
            pltpu.VMEM((CHUNK, 16), f32),
            pltpu.VMEM((16,), f32),
            pltpu.VMEM((ROWS_PER_TILE, 16), f32),
            pltpu.VMEM_SHARED((NPAD, 16), f32),
            pltpu.VMEM_SHARED((NPAD, 16), f32),
            pltpu.SemaphoreType.DMA,
            pltpu.SemaphoreType.DMA,
            pltpu.SemaphoreType.DMA,
        ],
    )(_sc2_body)
    u2, d2 = sc2(srcr, dstr, h2t, as2t, ad2t, m2)

    out = pl.pallas_call(
        _tcc_body,
        out_shape=jax.ShapeDtypeStruct((NPAD, 16), f32),
    )(u2, d2, b2)
    return out[:N]


# speedup vs baseline: 2.2635x; 1.0012x over previous
import functools

import jax
import jax.numpy as jnp
from jax import lax
from jax.experimental import pallas as pl
from jax.experimental.pallas import tpu as pltpu
from jax.experimental.pallas import tpu_sc as plsc

N = 10000
NPAD = 10112
E_RAW = 320000
E_TOT = E_RAW + N
NC, NS, NW = 2, 16, 32
CHUNK = 128
NCHUNK = 81
EPAD = NW * NCHUNK * CHUNK
ROWS_PER_TILE = NPAD // NS
Z1, Z2 = 312, 320

_MESH = dict(core_axis_name="c", subcore_axis_name="s", num_cores=NC,
             num_subcores=NS)


def _tca_body(x_ref, w1_ref, asm_ref, adm_ref,
              hg0_ref, hg1_ref, hg2_ref, hg3_ref, as1t_ref, ad1t_ref,
              m1_ref):
    h = jnp.dot(x_ref[...], w1_ref[...], preferred_element_type=jnp.float32)
    hg0_ref[...] = h[:, 0:64]
    hg1_ref[...] = h[:, 64:128]
    hg2_ref[...] = h[:, 128:192]
    hg3_ref[...] = h[:, 192:256]
    as1 = jnp.dot(h, asm_ref[...], preferred_element_type=jnp.float32)
    ad1 = jnp.dot(h, adm_ref[...], preferred_element_type=jnp.float32)
    as1t_ref[...] = jnp.concatenate([as1, as1], axis=1)
    ad1t_ref[...] = jnp.concatenate([ad1, ad1], axis=1)
    m = (jnp.max(as1, axis=0, keepdims=True)
         + jnp.max(ad1, axis=0, keepdims=True))
    m = jnp.where(m >= 0, m, 0.2 * m)
    m1_ref[...] = jnp.concatenate([m, m], axis=1)


TCB_BLOCKS = 8
TCB_B = NPAD // TCB_BLOCKS


def _tcb_body(u0_ref, u1_ref, u2_ref, u3_ref, d1_ref, b1_ref, w2_ref,
              r8_ref, as2m_ref, ad2m_ref, h2_ref, as2t_ref, ad2t_ref,
              m2_ref, mscr):
    i = pl.program_id(0)
    d8 = (d1_ref[0] + d1_ref[1])[:, :8]
    rec = 1.0 / (d8 + 1e-16)
    recx = jnp.dot(rec, r8_ref[...], preferred_element_type=jnp.float32)
    u = jnp.concatenate([u0_ref[0] + u0_ref[1], u1_ref[0] + u1_ref[1],
                         u2_ref[0] + u2_ref[1], u3_ref[0] + u3_ref[1]],
                        axis=1)
    h1 = u * recx + b1_ref[...]
    h1 = jnp.where(h1 > 0, h1, jnp.exp(jnp.minimum(h1, 0.0)) - 1.0)
    h2 = jnp.dot(h1, w2_ref[...], preferred_element_type=jnp.float32)
    h2_ref[...] = h2
    as2t = jnp.dot(h2, as2m_ref[...], preferred_element_type=jnp.float32)
    ad2t = jnp.dot(h2, ad2m_ref[...], preferred_element_type=jnp.float32)
    as2t_ref[...] = as2t
    ad2t_ref[...] = ad2t

    @pl.when(i == 0)
    def _():
        mscr[...] = jnp.full((2, 16), -jnp.inf, jnp.float32)

    bs = jnp.maximum(mscr[0:1, :], jnp.max(as2t, axis=0, keepdims=True))
    bd = jnp.maximum(mscr[1:2, :], jnp.max(ad2t, axis=0, keepdims=True))
    mscr[0:1, :] = bs
    mscr[1:2, :] = bd

    @pl.when(i == TCB_BLOCKS - 1)
    def _():
        m = mscr[0:1, :] + mscr[1:2, :]
        m2_ref[...] = jnp.where(m >= 0, m, 0.2 * m)


def _tcc_body(u2_ref, d2_ref, b2_ref, out_ref):
    out_ref[...] = ((u2_ref[0] + u2_ref[1])
                    / (d2_ref[0] + d2_ref[1] + 1e-16) + b2_ref[...])


def _sc1_body(srcr, dstr, as1t, ad1t, hg0, hg1, hg2, hg3, m1,
              u0o, u1o, u2o, u3o, d1o,
              sidx, didx, sA, sB, dA, dB, pbuf, hA, hB, m2v, zu, zd,
              accu, accd, asem, dsem, hsem, ssem):
    cid = lax.axis_index("c")
    sid = lax.axis_index("s")
    w = cid * NS + sid
    pltpu.sync_copy(srcr.at[w], sidx)
    pltpu.sync_copy(dstr.at[w], didx)
    pltpu.sync_copy(m1.at[0], m2v)
    mvec = m2v[...]
    zeros16 = jnp.zeros((16,), jnp.float32)

    def _zu(r, _):
        for k in range(4):
            zu[r, pl.ds(k * 16, 16)] = zeros16
        return 0
    lax.fori_loop(0, Z2, _zu, 0)

    def _zd(r, _):
        zd[r, :] = zeros16
        return 0
    lax.fori_loop(0, ROWS_PER_TILE, _zd, 0)

    r0 = sid * ROWS_PER_TILE
    pltpu.sync_copy(zu.at[pl.ds(0, Z1)], accu.at[pl.ds(r0, Z1)])
    pltpu.sync_copy(zu, accu.at[pl.ds(r0 + Z1, Z2)])
    pltpu.sync_copy(zd, accd.at[pl.ds(r0, ROWS_PER_TILE)])
    plsc.subcore_barrier()

    for g in range(4):
        hg = (hg0, hg1, hg2, hg3)[g]

        def _chunk(j, _):
            @pl.when(j > 0)
            def _():
                pltpu.make_async_copy(hA, accu.at[didx.at[j - 1]],
                                      ssem).wait()
            hd = pltpu.async_copy(hg.at[sidx.at[j]], hA, hsem)
            pltpu.sync_copy(as1t.at[sidx.at[j]], sA)
            pltpu.sync_copy(ad1t.at[didx.at[j]], dA)

            def _pcalc(c, _c):
                e = sA[c, :] + dA[c, :]
                e = jnp.where(e >= 0, e, 0.2 * e)
                pbuf[c, :] = jnp.exp(e - mvec)
                return 0
            lax.fori_loop(0, CHUNK, _pcalc, 0)
            if g == 0:
                pltpu.sync_copy(pbuf, accd.at[didx.at[j]], add=True)
            hd.wait()

            def _mul(c, _c):
                prow = pbuf[c, :]
                for h in range(2):
                    vec = jnp.full((16,), prow[g * 2 + h], jnp.float32)
                    for half in range(2):
                        sl = pl.ds(h * 32 + half * 16, 16)
                        hA[c, sl] = hA[c, sl] * vec
                return 0
            lax.fori_loop(0, CHUNK, _mul, 0)
            pltpu.async_copy(hA, accu.at[didx.at[j]], ssem, add=True)
            return 0
        lax.fori_loop(0, NCHUNK, _chunk, 0)
        pltpu.make_async_copy(hA, accu.at[didx.at[NCHUNK - 1]], ssem).wait()
        plsc.subcore_barrier()

        uo = (u0o, u1o, u2o, u3o)[g]
        pltpu.sync_copy(accu.at[pl.ds(r0, Z1)],
                        uo.at[cid, pl.ds(r0, Z1)])
        pltpu.sync_copy(accu.at[pl.ds(r0 + Z1, Z2)],
                        uo.at[cid, pl.ds(r0 + Z1, Z2)])
        if g == 0:
            pltpu.sync_copy(accd.at[pl.ds(r0, ROWS_PER_TILE)],
                            d1o.at[cid, pl.ds(r0, ROWS_PER_TILE)])
        if g < 3:
            pltpu.sync_copy(zu.at[pl.ds(0, Z1)], accu.at[pl.ds(r0, Z1)])
            pltpu.sync_copy(zu, accu.at[pl.ds(r0 + Z1, Z2)])
            plsc.subcore_barrier()


def _sc2_body(srcr, dstr, h2t, as2t, ad2t, m2,
              u2o, d2o,
              sidx, didx, sA, sB, dA, dB, pbuf, hA, hB, m2v, zd,
              accu, accd, asem, dsem, hsem, ssem):
    cid = lax.axis_index("c")
    sid = lax.axis_index("s")
    w = cid * NS + sid
    pltpu.sync_copy(srcr.at[w], sidx)
    pltpu.sync_copy(dstr.at[w], didx)
    pltpu.sync_copy(m2.at[0], m2v)
    mvec = m2v[...]
    zeros16 = jnp.zeros((16,), jnp.float32)

    def _zd(r, _):
        zd[r, :] = zeros16
        return 0
    lax.fori_loop(0, ROWS_PER_TILE, _zd, 0)

    r0 = sid * ROWS_PER_TILE
    pltpu.sync_copy(zd, accu.at[pl.ds(r0, ROWS_PER_TILE)])
    pltpu.sync_copy(zd, accd.at[pl.ds(r0, ROWS_PER_TILE)])
    plsc.subcore_barrier()

    def _chunk(j, _):
        @pl.when(j > 0)
        def _():
            pltpu.make_async_copy(hA, accu.at[didx.at[j - 1]], ssem).wait()
        hd = pltpu.async_copy(h2t.at[sidx.at[j]], hA, hsem)
        pltpu.sync_copy(as2t.at[sidx.at[j]], sA)
        pltpu.sync_copy(ad2t.at[didx.at[j]], dA)

        def _pcalc(c, _c):
            e = sA[c, :] + dA[c, :]
            e = jnp.where(e >= 0, e, 0.2 * e)
            pbuf[c, :] = jnp.exp(e - mvec)
            return 0
        lax.fori_loop(0, CHUNK, _pcalc, 0)
        pltpu.sync_copy(pbuf, accd.at[didx.at[j]], add=True)
        hd.wait()

        def _mul(c, _c):
            hA[c, :] = hA[c, :] * pbuf[c, :]
            return 0
        lax.fori_loop(0, CHUNK, _mul, 0)
        pltpu.async_copy(hA, accu.at[didx.at[j]], ssem, add=True)
        return 0
    lax.fori_loop(0, NCHUNK, _chunk, 0)
    pltpu.make_async_copy(hA, accu.at[didx.at[NCHUNK - 1]], ssem).wait()
    plsc.subcore_barrier()

    pltpu.sync_copy(accu.at[pl.ds(r0, ROWS_PER_TILE)],
                    u2o.at[cid, pl.ds(r0, ROWS_PER_TILE)])
    pltpu.sync_copy(accd.at[pl.ds(r0, ROWS_PER_TILE)],
                    d2o.at[cid, pl.ds(r0, ROWS_PER_TILE)])


def kernel(x, edge_index, W1, a_src1, a_dst1, b1, W2, a_src2, a_dst2, b2):
    f32 = jnp.float32
    loop = jnp.arange(N, dtype=jnp.int32)
    padi = jnp.full((EPAD - E_TOT,), N, jnp.int32)
    src = jnp.concatenate([edge_index[0].astype(jnp.int32), loop, padi])
    dst = jnp.concatenate([edge_index[1].astype(jnp.int32), loop, padi])
    srcr = src.reshape(NW, NCHUNK, CHUNK)
    dstr = dst.reshape(NW, NCHUNK, CHUNK)

    x_pad = jnp.zeros((NPAD, 128), f32).at[:N].set(x)
    eye8 = jnp.eye(8, dtype=f32)
    asm = (a_src1[:, :, None] * eye8[:, None, :]).reshape(256, 8)
    adm = (a_dst1[:, :, None] * eye8[:, None, :]).reshape(256, 8)
    r8 = jnp.kron(eye8, jnp.ones((1, 32), f32))
    as2m = jnp.broadcast_to(a_src2.reshape(16, 1), (16, 16))
    ad2m = jnp.broadcast_to(a_dst2.reshape(16, 1), (16, 16))

    hg0, hg1, hg2, hg3, as1t, ad1t, m1 = pl.pallas_call(
        _tca_body,
        out_shape=[
            jax.ShapeDtypeStruct((NPAD, 64), f32),
            jax.ShapeDtypeStruct((NPAD, 64), f32),
            jax.ShapeDtypeStruct((NPAD, 64), f32),
            jax.ShapeDtypeStruct((NPAD, 64), f32),
            jax.ShapeDtypeStruct((NPAD, 16), f32),
            jax.ShapeDtypeStruct((NPAD, 16), f32),
            jax.ShapeDtypeStruct((1, 16), f32),
        ],
    )(x_pad, W1, asm, adm)

    sc1 = functools.partial(
        pl.kernel,
        out_type=[
            jax.ShapeDtypeStruct((NC, NPAD, 64), f32),
            jax.ShapeDtypeStruct((NC, NPAD, 64), f32),
            jax.ShapeDtypeStruct((NC, NPAD, 64), f32),
            jax.ShapeDtypeStruct((NC, NPAD, 64), f32),
            jax.ShapeDtypeStruct((NC, NPAD, 16), f32),
        ],
        mesh=plsc.VectorSubcoreMesh(**_MESH),
        compiler_params=pltpu.CompilerParams(use_tc_tiling_on_sc=False),
        scratch_types=[
            pltpu.VMEM((NCHUNK, CHUNK), jnp.int32),
            pltpu.VMEM((NCHUNK, CHUNK), jnp.int32),
            pltpu.VMEM((CHUNK, 16), f32),
            pltpu.VMEM((CHUNK, 16), f32),
            pltpu.VMEM((CHUNK, 16), f32),
            pltpu.VMEM((CHUNK, 16), f32),
            pltpu.VMEM((CHUNK, 16), f32),
            pltpu.VMEM((CHUNK, 64), f32),
            pltpu.VMEM((CHUNK, 64), f32),
            pltpu.VMEM((16,), f32),
            pltpu.VMEM((Z2, 64), f32),
            pltpu.VMEM((ROWS_PER_TILE, 16), f32),
            pltpu.VMEM_SHARED((NPAD, 64), f32),
            pltpu.VMEM_SHARED((NPAD, 16), f32),
            pltpu.SemaphoreType.DMA,
            pltpu.SemaphoreType.DMA,
            pltpu.SemaphoreType.DMA,
            pltpu.SemaphoreType.DMA,
        ],
    )(_sc1_body)
    u0, u1, u2, u3, d1 = sc1(srcr, dstr, as1t, ad1t, hg0, hg1, hg2, hg3,
                             m1)

    ublock = pl.BlockSpec((NC, TCB_B, 64), lambda i: (0, i, 0))
    h2t, as2t, ad2t, m2 = pl.pallas_call(
        _tcb_body,
        grid=(TCB_BLOCKS,),
        in_specs=[
            ublock, ublock, ublock, ublock,
            pl.BlockSpec((NC, TCB_B, 16), lambda i: (0, i, 0)),
            pl.BlockSpec((256,), lambda i: (0,)),
            pl.BlockSpec((256, 16), lambda i: (0, 0)),
            pl.BlockSpec((8, 256), lambda i: (0, 0)),
            pl.BlockSpec((16, 16), lambda i: (0, 0)),
            pl.BlockSpec((16, 16), lambda i: (0, 0)),
        ],
        out_specs=[
            pl.BlockSpec((TCB_B, 16), lambda i: (i, 0)),
            pl.BlockSpec((TCB_B, 16), lambda i: (i, 0)),
            pl.BlockSpec((TCB_B, 16), lambda i: (i, 0)),
            pl.BlockSpec((1, 16), lambda i: (0, 0)),
        ],
        out_shape=[
            jax.ShapeDtypeStruct((NPAD, 16), f32),
            jax.ShapeDtypeStruct((NPAD, 16), f32),
            jax.ShapeDtypeStruct((NPAD, 16), f32),
            jax.ShapeDtypeStruct((1, 16), f32),
        ],
        scratch_shapes=[pltpu.VMEM((2, 16), f32)],
    )(u0, u1, u2, u3, d1, b1, W2, r8, as2m, ad2m)

    sc2 = functools.partial(
        pl.kernel,
        out_type=[
            jax.ShapeDtypeStruct((NC, NPAD, 16), f32),
            jax.ShapeDtypeStruct((NC, NPAD, 16), f32),
        ],
        mesh=plsc.VectorSubcoreMesh(**_MESH),
        compiler_params=pltpu.CompilerParams(use_tc_tiling_on_sc=False),
        scratch_types=[
            pltpu.VMEM((NCHUNK, CHUNK), jnp.int32),
            pltpu.VMEM((NCHUNK, CHUNK), jnp.int32),
            pltpu.VMEM((CHUNK, 16), f32),
            pltpu.VMEM((CHUNK, 16), f32),
            pltpu.VMEM((CHUNK, 16), f32),
            pltpu.VMEM((CHUNK, 16), f32),
            pltpu.VMEM((CHUNK, 16), f32),
            pltpu.VMEM((CHUNK, 16), f32),
            pltpu.VMEM((CHUNK, 16), f32),
            pltpu.VMEM((16,), f32),
            pltpu.VMEM((ROWS_PER_TILE, 16), f32),
            pltpu.VMEM_SHARED((NPAD, 16), f32),
            pltpu.VMEM_SHARED((NPAD, 16), f32),
            pltpu.SemaphoreType.DMA,
            pltpu.SemaphoreType.DMA,
            pltpu.SemaphoreType.DMA,
            pltpu.SemaphoreType.DMA,
        ],
    )(_sc2_body)
    u2, d2 = sc2(srcr, dstr, h2t, as2t, ad2t, m2)

    out = pl.pallas_call(
        _tcc_body,
        out_shape=jax.ShapeDtypeStruct((NPAD, 16), f32),
    )(u2, d2, b2)
    return out[:N]
